# baseline (speedup 1.0000x reference)
import jax
import jax.numpy as jnp
from jax.experimental import pallas as pl

K = 4
BLK = 1024


def _body(x0_ref, x1_ref, x2_ref, x3_ref, wt_ref, o_ref):
    wt = wt_ref[...]
    for k, xr in enumerate((x0_ref, x1_ref, x2_ref, x3_ref)):
        o_ref[k] = jnp.dot(xr[...], wt, preferred_element_type=jnp.float32)


def kernel(x, W):
    T, D = x.shape
    E = W.shape[0]
    Wt = W.T
    C = T // K
    xs = x.reshape(K, C, D)
    chunks = [xs[k] for k in range(K)]
    grid = (C // BLK,)
    out = pl.pallas_call(
        _body,
        grid=grid,
        in_specs=[pl.BlockSpec((BLK, D), lambda i: (i, 0)) for _ in range(K)]
        + [pl.BlockSpec((D, E), lambda i: (0, 0))],
        out_specs=pl.BlockSpec((K, BLK, E), lambda i: (0, i, 0)),
        out_shape=jax.ShapeDtypeStruct((K, C, E), jnp.float32),
    )(*chunks, Wt)
    return out.reshape(T, E)


# speedup vs baseline: 2.3508x; 2.3508x over previous
import jax
import jax.numpy as jnp
from jax.experimental import pallas as pl

K = 4
BLK = 1024


def _body(x0_ref, x1_ref, x2_ref, x3_ref, wt_ref, o_ref):
    wt = wt_ref[...]
    for k, xr in enumerate((x0_ref, x1_ref, x2_ref, x3_ref)):
        o_ref[k] = jnp.dot(xr[...], wt, preferred_element_type=jnp.float32)


def kernel(x, W):
    T, D = x.shape
    E = W.shape[0]
    Wt = W.T
    C = T // K
    steps = C // BLK
    in_specs = [
        pl.BlockSpec((BLK, D), lambda i, k=k: (k * steps + i, 0))
        for k in range(K)
    ] + [pl.BlockSpec((D, E), lambda i: (0, 0))]
    out = pl.pallas_call(
        _body,
        grid=(steps,),
        in_specs=in_specs,
        out_specs=pl.BlockSpec((K, BLK, E), lambda i: (0, i, 0)),
        out_shape=jax.ShapeDtypeStruct((K, C, E), jnp.float32),
    )(x, x, x, x, Wt)
    return out.reshape(T, E)
